# baseline (speedup 1.0000x reference)
import functools

import jax
import jax.numpy as jnp
from jax import lax
from jax.experimental import pallas as pl
from jax.experimental.pallas import tpu as pltpu
from jax.experimental.pallas import tpu_sc as plsc

_N = 10000
_DI = 128
_E = 320000
_NSC = 2
_NSUB = 16
_NTILES = _NSC * _NSUB
_NROWS = 10016
_RPT = _NROWS // _NSUB
_CH1, _N1A, _N1B = 48, 260, 160
_CH2, _N2A, _N2B = 128, 100, 60
_NB = 4


def _make_segsum(d, ch, n0, n1, seeded):
  mesh = plsc.VectorSubcoreMesh(core_axis_name="c", subcore_axis_name="s")
  nz = _RPT // ch
  rz = _RPT - nz * ch
  assert n0 % _NB == 0 and n1 % _NB == 0 and n1 <= n0

  @functools.partial(
      pl.kernel,
      mesh=mesh,
      compiler_params=pltpu.CompilerParams(use_tc_tiling_on_sc=False),
      out_type=jax.ShapeDtypeStruct((_NSC, _NROWS, d), jnp.float32),
      scratch_types=[
          pltpu.VMEM((2, n0, ch), jnp.int32),
          [pltpu.VMEM((ch, d), jnp.float32)] * _NB,
          pltpu.VMEM_SHARED((_NROWS, d), jnp.float32),
          [pltpu.SemaphoreType.DMA] * _NB,
          [pltpu.SemaphoreType.DMA] * _NB,
          pltpu.SemaphoreType.DMA,
      ],
  )
  def segsum(table, edges, seed, out, idx, rows, acc, gsem, ssem, zsem):
    c = lax.axis_index("c")
    s = lax.axis_index("s")
    wid = c * _NSUB + s
    nc = jnp.where(c == 0, n0, n1)
    r0 = s * _RPT
    sidx = idx.at[0]
    didx = idx.at[1]
    pltpu.sync_copy(edges.at[0, wid], sidx)
    pltpu.sync_copy(edges.at[1, wid], didx)
    for b in range(2):
      pltpu.async_copy(table.at[sidx.at[b]], rows[b], gsem[b])
    if seeded:
      @pl.when(c == 0)
      def _():
        pltpu.sync_copy(seed.at[pl.ds(r0, _RPT)], acc.at[pl.ds(r0, _RPT)])

    @pl.when((c != 0) if seeded else (c == c))
    def _():
      z16 = jnp.zeros((16,), jnp.float32)

      def zrow(i, carry):
        for k in range(d // 16):
          rows[2][i, pl.ds(16 * k, 16)] = z16
        return carry

      lax.fori_loop(0, ch, zrow, 0)
      for q in range(nz):
        pltpu.async_copy(rows[2], acc.at[pl.ds(r0 + q * ch, ch)], zsem)
      if rz:
        pltpu.async_copy(
            rows[2].at[pl.ds(0, rz)], acc.at[pl.ds(r0 + nz * ch, rz)], zsem)
      for q in range(nz):
        pltpu.make_async_copy(rows[2], acc.at[pl.ds(r0 + q * ch, ch)],
                              zsem).wait()
      if rz:
        pltpu.make_async_copy(
            rows[2].at[pl.ds(0, rz)], acc.at[pl.ds(r0 + nz * ch, rz)],
            zsem).wait()

    plsc.subcore_barrier()

    def grp(k, carry):
      j0 = _NB * k
      for b in range(_NB):
        j = j0 + b
        pltpu.make_async_copy(table.at[sidx.at[j]], rows[b], gsem[b]).wait()
        pltpu.async_copy(rows[b], acc.at[didx.at[j]], ssem[b], add=True)
        b2 = (b + 2) % _NB

        @pl.when(j >= 2)
        def _():
          pltpu.make_async_copy(
              rows[b2], acc.at[didx.at[j - 2]], ssem[b2]).wait()

        @pl.when(j + 2 < nc)
        def _():
          pltpu.async_copy(table.at[sidx.at[j + 2]], rows[b2], gsem[b2])
      return carry

    lax.fori_loop(0, nc // _NB, grp, 0)
    for off, b in ((2, 2), (1, 3)):
      pltpu.make_async_copy(rows[b], acc.at[didx.at[nc - off]], ssem[b]).wait()
    plsc.subcore_barrier()
    pltpu.sync_copy(acc.at[pl.ds(r0, _RPT)], out.at[c, pl.ds(r0, _RPT)])

  return segsum


_SEGSUM128 = _make_segsum(_DI, _CH1, _N1A, _N1B, seeded=False)
_SEGSUM16 = _make_segsum(16, _CH2, _N2A, _N2B, seeded=True)

_BM = 2504


def _dense_mid(parts, xf, w1r, b1, w1o, w2r, w2o, b2):
  def body(p0, p1, xb, w1r_r, b1_r, w1o_r, w2r_r, w2o_r, b2_r, p2_o, r2_o):
    agg = p0[0] + p1[0]
    h = jnp.dot(agg, w1r_r[...], preferred_element_type=jnp.float32)
    h += b1_r[...]
    h += jnp.dot(xb[...], w1o_r[...], preferred_element_type=jnp.float32)
    h = jnp.maximum(h, 0.0)
    p2_o[...] = jnp.dot(h, w2r_r[...], preferred_element_type=jnp.float32)
    r2_o[...] = jnp.dot(h, w2o_r[...], preferred_element_type=jnp.float32) + b2_r[...]

  return pl.pallas_call(
      body,
      grid=(_NROWS // _BM,),
      in_specs=[
          pl.BlockSpec((1, _BM, _DI), lambda i: (0, i, 0)),
          pl.BlockSpec((1, _BM, _DI), lambda i: (1, i, 0)),
          pl.BlockSpec((_BM, _DI), lambda i: (i, 0)),
          pl.BlockSpec((_DI, _DI), lambda i: (0, 0)),
          pl.BlockSpec((1, _DI), lambda i: (0, 0)),
          pl.BlockSpec((_DI, _DI), lambda i: (0, 0)),
          pl.BlockSpec((_DI, 16), lambda i: (0, 0)),
          pl.BlockSpec((_DI, 16), lambda i: (0, 0)),
          pl.BlockSpec((1, 16), lambda i: (0, 0)),
      ],
      out_specs=[
          pl.BlockSpec((_BM, 16), lambda i: (i, 0)),
          pl.BlockSpec((_BM, 16), lambda i: (i, 0)),
      ],
      out_shape=[
          jax.ShapeDtypeStruct((_NROWS, 16), jnp.float32),
          jax.ShapeDtypeStruct((_NROWS, 16), jnp.float32),
      ],
  )(parts, parts, xf, w1r, b1, w1o, w2r, w2o, b2)


def _final_add(parts2):
  def body(q0, q1, o):
    o[...] = q0[0] + q1[0]

  return pl.pallas_call(
      body,
      grid=(_NROWS // _BM,),
      in_specs=[
          pl.BlockSpec((1, _BM, 16), lambda i: (0, i, 0)),
          pl.BlockSpec((1, _BM, 16), lambda i: (1, i, 0)),
      ],
      out_specs=pl.BlockSpec((_BM, 16), lambda i: (i, 0)),
      out_shape=jax.ShapeDtypeStruct((_N, 16), jnp.float32),
  )(parts2, parts2)


def _pack_edges(adj, ch, n0, n1):
  e0 = _NSUB * n0 * ch
  npad = e0 + _NSUB * n1 * ch - _E
  fill = jnp.stack([
      jnp.zeros((npad,), jnp.int32),
      _N + (jnp.arange(npad, dtype=jnp.int32) % (_NROWS - _N)),
  ])
  padded = jnp.concatenate([adj, fill], axis=1)
  c0 = padded[:, :e0].reshape(2, _NSUB, n0, ch)
  c1 = padded[:, e0:].reshape(2, _NSUB, n1, ch)
  c1 = jnp.pad(c1, ((0, 0), (0, 0), (0, n0 - n1), (0, 0)))
  return jnp.concatenate([c0, c1], axis=1)


def kernel(adj_est, x, W1_rel, b1_rel, W1_root, W2_rel, b2_rel, W2_root):
  xf = x.reshape(_N, _DI)
  edges1 = _pack_edges(adj_est, _CH1, _N1A, _N1B)
  edges2 = _pack_edges(adj_est, _CH2, _N2A, _N2B)

  parts1 = _SEGSUM128(xf, edges1, xf)

  w2r = jnp.pad(W2_rel, ((0, 0), (0, 16 - W2_rel.shape[1])))
  w2o = jnp.pad(W2_root, ((0, 0), (0, 16 - W2_root.shape[1])))
  b2 = jnp.pad(b2_rel, (0, 16 - b2_rel.shape[0])).reshape(1, 16)
  p2, r2 = _dense_mid(parts1, xf, W1_rel, b1_rel.reshape(1, _DI), W1_root,
                      w2r, w2o, b2)

  parts2 = _SEGSUM16(p2, edges2, r2)

  out16 = _final_add(parts2)
  return out16[:, :7].reshape(1, _N, 7)


# speedup vs baseline: 1.3151x; 1.3151x over previous
import functools

import jax
import jax.numpy as jnp
from jax import lax
from jax.experimental import pallas as pl
from jax.experimental.pallas import tpu as pltpu
from jax.experimental.pallas import tpu_sc as plsc

_N = 10000
_DI = 128
_E = 320000
_NSC = 2
_NSUB = 16
_NTILES = _NSC * _NSUB
_NROWS = 10016
_RPT = _NROWS // _NSUB
_CH1, _N1A, _N1B = 50, 200, 200
_CH2, _N2A, _N2B = 125, 104, 56
_NB = 4


def _make_segsum(d, ch, n0, n1, seeded):
  mesh = plsc.VectorSubcoreMesh(core_axis_name="c", subcore_axis_name="s")
  nz = _RPT // ch
  rz = _RPT - nz * ch
  assert n0 % _NB == 0 and n1 % _NB == 0 and n1 <= n0

  @functools.partial(
      pl.kernel,
      mesh=mesh,
      compiler_params=pltpu.CompilerParams(use_tc_tiling_on_sc=False),
      out_type=jax.ShapeDtypeStruct((_NSC, _NROWS, d), jnp.float32),
      scratch_types=[
          pltpu.VMEM((2, n0, ch), jnp.int32),
          [pltpu.VMEM((ch, d), jnp.float32)] * _NB,
          pltpu.VMEM_SHARED((_NROWS, d), jnp.float32),
          [pltpu.SemaphoreType.DMA] * _NB,
          [pltpu.SemaphoreType.DMA] * _NB,
          pltpu.SemaphoreType.DMA,
      ],
  )
  def segsum(table, edges, seed, out, idx, rows, acc, gsem, ssem, zsem):
    c = lax.axis_index("c")
    s = lax.axis_index("s")
    nc = jnp.where(c == 0, n0, n1)
    r0 = s * _RPT
    sidx = idx.at[0]
    didx = idx.at[1]

    @pl.when(c == 0)
    def _():
      for a in range(2):
        pltpu.sync_copy(edges.at[a, pl.ds(s * n0, n0)], idx.at[a])

    @pl.when(c != 0)
    def _():
      for a in range(2):
        pltpu.sync_copy(edges.at[a, pl.ds(_NSUB * n0 + s * n1, n1)],
                        idx.at[a, pl.ds(0, n1)])
    for b in range(2):
      pltpu.async_copy(table.at[sidx.at[b]], rows[b], gsem[b])
    if seeded:
      @pl.when(c == 0)
      def _():
        pltpu.sync_copy(seed.at[pl.ds(r0, _RPT)], acc.at[pl.ds(r0, _RPT)])

    @pl.when((c != 0) if seeded else (c == c))
    def _():
      z16 = jnp.zeros((16,), jnp.float32)

      def zrow(i, carry):
        for k in range(d // 16):
          rows[2][i, pl.ds(16 * k, 16)] = z16
        return carry

      lax.fori_loop(0, ch, zrow, 0)
      for q in range(nz):
        pltpu.async_copy(rows[2], acc.at[pl.ds(r0 + q * ch, ch)], zsem)
      if rz:
        pltpu.async_copy(
            rows[2].at[pl.ds(0, rz)], acc.at[pl.ds(r0 + nz * ch, rz)], zsem)
      for q in range(nz):
        pltpu.make_async_copy(rows[2], acc.at[pl.ds(r0 + q * ch, ch)],
                              zsem).wait()
      if rz:
        pltpu.make_async_copy(
            rows[2].at[pl.ds(0, rz)], acc.at[pl.ds(r0 + nz * ch, rz)],
            zsem).wait()

    plsc.subcore_barrier()

    def grp(k, carry):
      j0 = _NB * k
      for b in range(_NB):
        j = j0 + b
        pltpu.make_async_copy(table.at[sidx.at[j]], rows[b], gsem[b]).wait()
        pltpu.async_copy(rows[b], acc.at[didx.at[j]], ssem[b], add=True)
        b2 = (b + 2) % _NB

        @pl.when(j >= 2)
        def _():
          pltpu.make_async_copy(
              rows[b2], acc.at[didx.at[j - 2]], ssem[b2]).wait()

        @pl.when(j + 2 < nc)
        def _():
          pltpu.async_copy(table.at[sidx.at[j + 2]], rows[b2], gsem[b2])
      return carry

    lax.fori_loop(0, nc // _NB, grp, 0)
    for off, b in ((2, 2), (1, 3)):
      pltpu.make_async_copy(rows[b], acc.at[didx.at[nc - off]], ssem[b]).wait()
    plsc.subcore_barrier()
    pltpu.sync_copy(acc.at[pl.ds(r0, _RPT)], out.at[c, pl.ds(r0, _RPT)])

  return segsum


_SEGSUM128 = _make_segsum(_DI, _CH1, _N1A, _N1B, seeded=False)
_SEGSUM16 = _make_segsum(16, _CH2, _N2A, _N2B, seeded=True)

_BM = 2504


def _dense_mid(parts, xf, w1r, b1, w1o, w2r, w2o, b2):
  def body(p0, p1, xb, w1r_r, b1_r, w1o_r, w2r_r, w2o_r, b2_r, p2_o, r2_o):
    agg = p0[0] + p1[0]
    h = jnp.dot(agg, w1r_r[...], preferred_element_type=jnp.float32)
    h += b1_r[...]
    h += jnp.dot(xb[...], w1o_r[...], preferred_element_type=jnp.float32)
    h = jnp.maximum(h, 0.0)
    p2_o[...] = jnp.dot(h, w2r_r[...], preferred_element_type=jnp.float32)
    r2_o[...] = jnp.dot(h, w2o_r[...], preferred_element_type=jnp.float32) + b2_r[...]

  return pl.pallas_call(
      body,
      grid=(_NROWS // _BM,),
      in_specs=[
          pl.BlockSpec((1, _BM, _DI), lambda i: (0, i, 0)),
          pl.BlockSpec((1, _BM, _DI), lambda i: (1, i, 0)),
          pl.BlockSpec((_BM, _DI), lambda i: (i, 0)),
          pl.BlockSpec((_DI, _DI), lambda i: (0, 0)),
          pl.BlockSpec((1, _DI), lambda i: (0, 0)),
          pl.BlockSpec((_DI, _DI), lambda i: (0, 0)),
          pl.BlockSpec((_DI, 16), lambda i: (0, 0)),
          pl.BlockSpec((_DI, 16), lambda i: (0, 0)),
          pl.BlockSpec((1, 16), lambda i: (0, 0)),
      ],
      out_specs=[
          pl.BlockSpec((_BM, 16), lambda i: (i, 0)),
          pl.BlockSpec((_BM, 16), lambda i: (i, 0)),
      ],
      out_shape=[
          jax.ShapeDtypeStruct((_NROWS, 16), jnp.float32),
          jax.ShapeDtypeStruct((_NROWS, 16), jnp.float32),
      ],
  )(parts, parts, xf, w1r, b1, w1o, w2r, w2o, b2)


def _final_add(parts2):
  def body(q0, q1, o):
    o[...] = q0[0] + q1[0]

  return pl.pallas_call(
      body,
      grid=(_NROWS // _BM,),
      in_specs=[
          pl.BlockSpec((1, _BM, 16), lambda i: (0, i, 0)),
          pl.BlockSpec((1, _BM, 16), lambda i: (1, i, 0)),
      ],
      out_specs=pl.BlockSpec((_BM, 16), lambda i: (i, 0)),
      out_shape=jax.ShapeDtypeStruct((_N, 16), jnp.float32),
  )(parts2, parts2)


def kernel(adj_est, x, W1_rel, b1_rel, W1_root, W2_rel, b2_rel, W2_root):
  xf = x.reshape(_N, _DI)
  edges1 = adj_est.reshape(2, _E // _CH1, _CH1)
  edges2 = adj_est.reshape(2, _E // _CH2, _CH2)

  parts1 = _SEGSUM128(xf, edges1, xf)

  w2r = jnp.pad(W2_rel, ((0, 0), (0, 16 - W2_rel.shape[1])))
  w2o = jnp.pad(W2_root, ((0, 0), (0, 16 - W2_root.shape[1])))
  b2 = jnp.pad(b2_rel, (0, 16 - b2_rel.shape[0])).reshape(1, 16)
  p2, r2 = _dense_mid(parts1, xf, W1_rel, b1_rel.reshape(1, _DI), W1_root,
                      w2r, w2o, b2)

  parts2 = _SEGSUM16(p2, edges2, r2)

  out16 = _final_add(parts2)
  return out16[:, :7].reshape(1, _N, 7)


# speedup vs baseline: 1.4463x; 1.0998x over previous
import functools

import jax
import jax.numpy as jnp
from jax import lax
from jax.experimental import pallas as pl
from jax.experimental.pallas import tpu as pltpu
from jax.experimental.pallas import tpu_sc as plsc

_N = 10000
_DI = 128
_E = 320000
_NSC = 2
_NSUB = 16
_NTILES = _NSC * _NSUB
_NROWS = 10016
_RPT = _NROWS // _NSUB
_CH1, _N1A, _N1B = 50, 200, 200
_CH2, _N2A, _N2B = 125, 80, 80


def _make_segsum(d, ch, n0, n1, nb, fd, seeded):
  mesh = plsc.VectorSubcoreMesh(core_axis_name="c", subcore_axis_name="s")
  nz = _RPT // ch
  rz = _RPT - nz * ch
  assert n0 % nb == 0 and n1 % nb == 0 and n1 <= n0 and nb >= 2 * fd

  @functools.partial(
      pl.kernel,
      mesh=mesh,
      compiler_params=pltpu.CompilerParams(use_tc_tiling_on_sc=False),
      out_type=jax.ShapeDtypeStruct((_NSC, _NROWS, d), jnp.float32),
      scratch_types=[
          pltpu.VMEM((2, n0, ch), jnp.int32),
          [pltpu.VMEM((ch, d), jnp.float32)] * nb,
          pltpu.VMEM_SHARED((_NROWS, d), jnp.float32),
          [pltpu.SemaphoreType.DMA] * nb,
          [pltpu.SemaphoreType.DMA] * nb,
          pltpu.SemaphoreType.DMA,
      ],
  )
  def segsum(table, edges, seed, out, idx, rows, acc, gsem, ssem, zsem):
    c = lax.axis_index("c")
    s = lax.axis_index("s")
    nc = jnp.where(c == 0, n0, n1)
    r0 = s * _RPT
    sidx = idx.at[0]
    didx = idx.at[1]

    @pl.when(c == 0)
    def _():
      for a in range(2):
        pltpu.sync_copy(edges.at[a, pl.ds(s * n0, n0)], idx.at[a])

    @pl.when(c != 0)
    def _():
      for a in range(2):
        pltpu.sync_copy(edges.at[a, pl.ds(_NSUB * n0 + s * n1, n1)],
                        idx.at[a, pl.ds(0, n1)])
    for b in range(fd):
      pltpu.async_copy(table.at[sidx.at[b]], rows[b], gsem[b])
    if seeded:
      @pl.when(c == 0)
      def _():
        pltpu.sync_copy(seed.at[pl.ds(r0, _RPT)], acc.at[pl.ds(r0, _RPT)])

    @pl.when((c != 0) if seeded else (c == c))
    def _():
      z16 = jnp.zeros((16,), jnp.float32)
      zbuf = rows[fd]

      def zrow(i, carry):
        for k in range(d // 16):
          zbuf[i, pl.ds(16 * k, 16)] = z16
        return carry

      lax.fori_loop(0, ch, zrow, 0)
      for q in range(nz):
        pltpu.async_copy(zbuf, acc.at[pl.ds(r0 + q * ch, ch)], zsem)
      if rz:
        pltpu.async_copy(
            zbuf.at[pl.ds(0, rz)], acc.at[pl.ds(r0 + nz * ch, rz)], zsem)
      for q in range(nz):
        pltpu.make_async_copy(zbuf, acc.at[pl.ds(r0 + q * ch, ch)],
                              zsem).wait()
      if rz:
        pltpu.make_async_copy(
            zbuf.at[pl.ds(0, rz)], acc.at[pl.ds(r0 + nz * ch, rz)],
            zsem).wait()

    plsc.subcore_barrier()

    def grp(k, carry):
      j0 = nb * k
      for b in range(nb):
        j = j0 + b
        pltpu.make_async_copy(table.at[sidx.at[j]], rows[b], gsem[b]).wait()
        pltpu.async_copy(rows[b], acc.at[didx.at[j]], ssem[b], add=True)
        b2 = (b + fd) % nb

        @pl.when(j >= fd)
        def _():
          pltpu.make_async_copy(
              rows[b2], acc.at[didx.at[j - fd]], ssem[b2]).wait()

        @pl.when(j + fd < nc)
        def _():
          pltpu.async_copy(table.at[sidx.at[j + fd]], rows[b2], gsem[b2])
      return carry

    lax.fori_loop(0, nc // nb, grp, 0)
    for off in range(fd, 0, -1):
      b = (nb - off) % nb
      pltpu.make_async_copy(rows[b], acc.at[didx.at[nc - off]], ssem[b]).wait()
    plsc.subcore_barrier()
    pltpu.sync_copy(acc.at[pl.ds(r0, _RPT)], out.at[c, pl.ds(r0, _RPT)])

  return segsum


_SEGSUM128 = _make_segsum(_DI, _CH1, _N1A, _N1B, nb=4, fd=2, seeded=False)
_SEGSUM16 = _make_segsum(16, _CH2, _N2A, _N2B, nb=8, fd=4, seeded=True)

_BM = 2504


def _dense_mid(parts, xf, w1r, b1, w1o, w2r, w2o, b2):
  def body(p0, p1, xb, w1r_r, b1_r, w1o_r, w2r_r, w2o_r, b2_r, p2_o, r2_o):
    agg = p0[0] + p1[0]
    h = jnp.dot(agg, w1r_r[...], preferred_element_type=jnp.float32)
    h += b1_r[...]
    h += jnp.dot(xb[...], w1o_r[...], preferred_element_type=jnp.float32)
    h = jnp.maximum(h, 0.0)
    p2_o[...] = jnp.dot(h, w2r_r[...], preferred_element_type=jnp.float32)
    r2_o[...] = jnp.dot(h, w2o_r[...], preferred_element_type=jnp.float32) + b2_r[...]

  return pl.pallas_call(
      body,
      grid=(_NROWS // _BM,),
      in_specs=[
          pl.BlockSpec((1, _BM, _DI), lambda i: (0, i, 0)),
          pl.BlockSpec((1, _BM, _DI), lambda i: (1, i, 0)),
          pl.BlockSpec((_BM, _DI), lambda i: (i, 0)),
          pl.BlockSpec((_DI, _DI), lambda i: (0, 0)),
          pl.BlockSpec((1, _DI), lambda i: (0, 0)),
          pl.BlockSpec((_DI, _DI), lambda i: (0, 0)),
          pl.BlockSpec((_DI, 16), lambda i: (0, 0)),
          pl.BlockSpec((_DI, 16), lambda i: (0, 0)),
          pl.BlockSpec((1, 16), lambda i: (0, 0)),
      ],
      out_specs=[
          pl.BlockSpec((_BM, 16), lambda i: (i, 0)),
          pl.BlockSpec((_BM, 16), lambda i: (i, 0)),
      ],
      out_shape=[
          jax.ShapeDtypeStruct((_NROWS, 16), jnp.float32),
          jax.ShapeDtypeStruct((_NROWS, 16), jnp.float32),
      ],
  )(parts, parts, xf, w1r, b1, w1o, w2r, w2o, b2)


def _final_add(parts2):
  def body(q0, q1, o):
    o[...] = q0[0] + q1[0]

  return pl.pallas_call(
      body,
      grid=(_NROWS // _BM,),
      in_specs=[
          pl.BlockSpec((1, _BM, 16), lambda i: (0, i, 0)),
          pl.BlockSpec((1, _BM, 16), lambda i: (1, i, 0)),
      ],
      out_specs=pl.BlockSpec((_BM, 16), lambda i: (i, 0)),
      out_shape=jax.ShapeDtypeStruct((_N, 16), jnp.float32),
  )(parts2, parts2)


def kernel(adj_est, x, W1_rel, b1_rel, W1_root, W2_rel, b2_rel, W2_root):
  xf = x.reshape(_N, _DI)
  edges1 = adj_est.reshape(2, _E // _CH1, _CH1)
  edges2 = adj_est.reshape(2, _E // _CH2, _CH2)

  parts1 = _SEGSUM128(xf, edges1, xf)

  w2r = jnp.pad(W2_rel, ((0, 0), (0, 16 - W2_rel.shape[1])))
  w2o = jnp.pad(W2_root, ((0, 0), (0, 16 - W2_root.shape[1])))
  b2 = jnp.pad(b2_rel, (0, 16 - b2_rel.shape[0])).reshape(1, 16)
  p2, r2 = _dense_mid(parts1, xf, W1_rel, b1_rel.reshape(1, _DI), W1_root,
                      w2r, w2o, b2)

  parts2 = _SEGSUM16(p2, edges2, r2)

  out16 = _final_add(parts2)
  return out16[:, :7].reshape(1, _N, 7)
